# baseline (speedup 1.0000x reference)
import functools

import jax
import jax.numpy as jnp
from jax import lax
from jax.experimental import pallas as pl
from jax.experimental.pallas import tpu as pltpu
from jax.experimental.pallas import tpu_sc as plsc

_N = 100000
_Z = 64
_R = 16
_B = 4096
_W = _Z * _R // 2

_NW = 32
_BPW = _B // _NW


def _sc_gather(y, lrp, md):
    mesh = plsc.VectorSubcoreMesh(core_axis_name="c", subcore_axis_name="s")

    @functools.partial(
        pl.kernel,
        mesh=mesh,
        out_type=(
            jax.ShapeDtypeStruct((_B, _W), jnp.float32),
            jax.ShapeDtypeStruct((_B, 2 * _Z), jnp.float32),
        ),
        scratch_types=[
            pltpu.VMEM((_BPW,), jnp.int32),
            pltpu.VMEM((_BPW, _W), jnp.float32),
            pltpu.VMEM((_BPW, 2 * _Z), jnp.float32),
            pltpu.SemaphoreType.DMA,
            pltpu.SemaphoreType.DMA,
        ],
    )
    def k(y_hbm, lr_hbm, md_hbm, lrg_hbm, mdg_hbm, idx_v, rows_v, md_v,
          sem_a, sem_b):
        wid = lax.axis_index("s") * 2 + lax.axis_index("c")
        base = wid * _BPW
        pltpu.sync_copy(y_hbm.at[pl.ds(base, _BPW)], idx_v)
        cp_lr = pltpu.async_copy(lr_hbm.at[idx_v], rows_v, sem_a)
        cp_md = pltpu.async_copy(md_hbm.at[idx_v], md_v, sem_b)
        cp_md.wait()
        pltpu.sync_copy(md_v, mdg_hbm.at[pl.ds(base, _BPW)])
        cp_lr.wait()
        pltpu.sync_copy(rows_v, lrg_hbm.at[pl.ds(base, _BPW)])

    return k(y, lrp, md)


_BC = 256


def _tc_body(lrg_ref, mdg_ref, out_ref):
    wt = lax.bitcast_convert_type(lrg_ref[...], jnp.uint32).T
    ge = lax.bitcast_convert_type(wt << jnp.uint32(16), jnp.float32)
    go = lax.bitcast_convert_type(wt & jnp.uint32(0xFFFF0000), jnp.float32)
    g3e = ge.reshape(_Z, _R // 2, _BC)
    g3o = go.reshape(_Z, _R // 2, _BC)
    mdt = mdg_ref[...].T
    mu_t = mdt[0:_Z]
    sp = jax.nn.softplus(mdt[_Z:2 * _Z])
    out_ref[0] = mu_t
    for j in range(_Z):
        if j > 0:
            out_ref[1 + j, 0:j] = jnp.zeros((j, _BC), jnp.float32)
        out_ref[1 + j, j:j + 1] = sp[j:j + 1]
        if j < _Z - 1:
            prod = g3e[j + 1:] * g3e[j][None] + g3o[j + 1:] * g3o[j][None]
            out_ref[1 + j, j + 1:_Z] = prod.sum(axis=1)


def _tc_build(lrg, mdg):
    return pl.pallas_call(
        _tc_body,
        grid=(_B // _BC,),
        in_specs=[
            pl.BlockSpec((_BC, _W), lambda g: (g, 0)),
            pl.BlockSpec((_BC, 2 * _Z), lambda g: (g, 0)),
        ],
        out_specs=pl.BlockSpec((_Z + 1, _Z, _BC), lambda g: (0, 0, g)),
        out_shape=jax.ShapeDtypeStruct((_Z + 1, _Z, _B), jnp.float32),
    )(lrg, mdg)


def kernel(y, mu, low_rank, diag):
    lrp = lax.bitcast_convert_type(
        low_rank.astype(jnp.bfloat16).reshape(_N, _W, 2), jnp.float32)
    md = jnp.concatenate([mu, diag], axis=1)
    lrg, mdg = _sc_gather(y, lrp, md)
    out_t = _tc_build(lrg, mdg)
    return jnp.transpose(out_t, (2, 1, 0))


# speedup vs baseline: 3.6418x; 3.6418x over previous
import functools

import jax
import jax.numpy as jnp
from jax import lax
from jax.experimental import pallas as pl
from jax.experimental.pallas import tpu as pltpu
from jax.experimental.pallas import tpu_sc as plsc

_N = 100000
_Z = 64
_R = 16
_B = 4096

_NW = 32
_BPW = _B // _NW
_CH = 64


def _sc_gather(y, lr2, md):
    mesh = plsc.VectorSubcoreMesh(core_axis_name="c", subcore_axis_name="s")

    @functools.partial(
        pl.kernel,
        mesh=mesh,
        out_type=(
            jax.ShapeDtypeStruct((_B, _Z * _R), jnp.float32),
            jax.ShapeDtypeStruct((_B, 2 * _Z), jnp.float32),
        ),
        scratch_types=[
            pltpu.VMEM((_BPW,), jnp.int32),
            pltpu.VMEM((_CH, _Z * _R), jnp.float32),
            pltpu.VMEM((_BPW, 2 * _Z), jnp.float32),
            pltpu.SemaphoreType.DMA,
            pltpu.SemaphoreType.DMA,
        ],
    )
    def k(y_hbm, lr_hbm, md_hbm, lrg_hbm, mdg_hbm, idx_v, rows_v,
          md_v, sem_lr, sem_md):
        wid = lax.axis_index("s") * 2 + lax.axis_index("c")
        base = wid * _BPW
        pltpu.sync_copy(y_hbm.at[pl.ds(base, _BPW)], idx_v)
        cp_md = pltpu.async_copy(md_hbm.at[idx_v], md_v, sem_md)
        for c in range(_BPW // _CH):
            idx_c = idx_v.at[pl.ds(c * _CH, _CH)]
            pltpu.async_copy(lr_hbm.at[idx_c], rows_v, sem_lr).wait()
            pltpu.sync_copy(rows_v, lrg_hbm.at[pl.ds(base + c * _CH, _CH)])
        cp_md.wait()
        pltpu.sync_copy(md_v, mdg_hbm.at[pl.ds(base, _BPW)])

    return k(y, lr2, md)


_BC = 256


def _tc_body(lrg_ref, mdg_ref, out_ref):
    gt = lrg_ref[...].T
    gt3 = gt.reshape(_Z, _R, _BC)
    mdt = mdg_ref[...].T
    mu_t = mdt[0:_Z]
    sp = jax.nn.softplus(mdt[_Z:2 * _Z])
    out_ref[0] = mu_t
    for j in range(_Z):
        if j > 0:
            out_ref[1 + j, 0:j] = jnp.zeros((j, _BC), jnp.float32)
        out_ref[1 + j, j:j + 1] = sp[j:j + 1]
        if j < _Z - 1:
            pj = gt3[j]
            prod = gt3[j + 1:] * pj[None]
            out_ref[1 + j, j + 1:_Z] = prod.sum(axis=1)


def _tc_build(lrg, mdg):
    return pl.pallas_call(
        _tc_body,
        grid=(_B // _BC,),
        in_specs=[
            pl.BlockSpec((_BC, _Z * _R), lambda g: (g, 0)),
            pl.BlockSpec((_BC, 2 * _Z), lambda g: (g, 0)),
        ],
        out_specs=pl.BlockSpec((_Z + 1, _Z, _BC), lambda g: (0, 0, g)),
        out_shape=jax.ShapeDtypeStruct((_Z + 1, _Z, _B), jnp.float32),
    )(lrg, mdg)


def kernel(y, mu, low_rank, diag):
    lr2 = low_rank.reshape(_N, _Z * _R)
    md = jnp.concatenate([mu, diag], axis=1)
    lrg, mdg = _sc_gather(y, lr2, md)
    out_t = _tc_build(lrg, mdg)
    return jnp.transpose(out_t, (2, 1, 0))
